# baseline (speedup 1.0000x reference)
import jax
import jax.numpy as jnp
from jax.experimental import pallas as pl


def _bn2d_(x, g, b, eps=1e-5):
    m = x.mean(axis=0)
    v = x.var(axis=0)
    return (x - m) / jnp.sqrt(v + eps) * g + b


def _bn3d_(x, g, b, eps=1e-5):
    m = x.mean(axis=(0, 2), keepdims=True)
    v = x.var(axis=(0, 2), keepdims=True)
    return (x - m) / jnp.sqrt(v + eps) * g[None, :, None] + b[None, :, None]


def _gcn_(x, w, bias, alpha_tab, edge_index, edge_type, n):
    alp = alpha_tab[edge_type][:, 0]
    support = x @ w
    row = edge_index[0]
    col = edge_index[1]
    n_static = x.shape[0]
    out = jax.ops.segment_sum(alp[:, None] * support[col], row, num_segments=n_static)
    out = out + jax.ops.segment_sum(alp[:, None] * support[row], col, num_segments=n_static)
    return out + bias


def _sigmoid_pallas(x):
    def body(x_ref, o_ref):
        o_ref[...] = jax.nn.sigmoid(x_ref[...])
    return pl.pallas_call(
        body, out_shape=jax.ShapeDtypeStruct(x.shape, x.dtype))(x)


def kernel(e1, rel, X, edge_index, edge_type, num_nodes, emb_e, gc1_w, gc1_b, gc1_alpha, gc2_w, gc2_b, gc2_alpha, emb_rel, conv_w, conv_b, fc_w, fc_b, bn0_g, bn0_b, bn1_g, bn1_b, bn2_g, bn2_b, bn3_g, bn3_b, bn4_g, bn4_b):
    B = e1.shape[0]
    emb_initial = emb_e[X]
    x = _gcn_(emb_initial, gc1_w, gc1_b, gc1_alpha, edge_index, edge_type, num_nodes)
    x = jnp.tanh(_bn2d_(x, bn3_g, bn3_b))
    x = _gcn_(x, gc2_w, gc2_b, gc2_alpha, edge_index, edge_type, num_nodes)
    e1_embedded_all = jnp.tanh(_bn2d_(x, bn4_g, bn4_b))
    e1_embedded = e1_embedded_all[e1]
    rel_embedded = emb_rel[rel]
    stacked = jnp.concatenate([e1_embedded, rel_embedded], axis=1)
    x = _bn3d_(stacked, bn0_g, bn0_b)
    x = jax.lax.conv_general_dilated(x, conv_w, window_strides=(1,), padding=((2, 2),), dimension_numbers=('NCH', 'OIH', 'NCH'))
    x = x + conv_b[None, :, None]
    x = jax.nn.relu(_bn3d_(x, bn1_g, bn1_b))
    x = x.reshape(B, -1)
    x = x @ fc_w.T + fc_b
    x = jax.nn.relu(_bn2d_(x, bn2_g, bn2_b))
    logits = x @ e1_embedded_all.T
    return _sigmoid_pallas(logits)


# speedup vs baseline: 2.5612x; 2.5612x over previous
import functools
import jax
import jax.numpy as jnp
from jax import lax
from jax.experimental import pallas as pl
from jax.experimental.pallas import tpu as pltpu
from jax.experimental.pallas import tpu_sc as plsc

N_ENT_ = 10000
NPAD_ = 10240
D_ = 200
DLO_ = 128
DHALF_ = 100
E_ = 320000
CHUNK_ = 80
STRIPE_ = E_ // 16
NCHUNK_ = STRIPE_ // CHUNK_
ZROWS_ = 128
ROWS_PER_TILE_ = NPAD_ // 16
RB_ = 1280
NRB_ = NPAD_ // RB_
EPS_ = 1e-5


def _sc_agg(sup_lo, sup_hi, row, col, etype, arows):
    mesh = plsc.VectorSubcoreMesh(core_axis_name="c", subcore_axis_name="s")

    @functools.partial(
        pl.kernel,
        mesh=mesh,
        out_type=[
            jax.ShapeDtypeStruct((NPAD_, DLO_), jnp.float32),
            jax.ShapeDtypeStruct((NPAD_, DLO_), jnp.float32),
        ],
        scratch_types=[
            pltpu.VMEM((CHUNK_, DLO_), jnp.float32),
            pltpu.VMEM((CHUNK_,), jnp.int32),
            pltpu.VMEM((CHUNK_,), jnp.int32),
            pltpu.VMEM((CHUNK_,), jnp.int32),
            pltpu.VMEM((CHUNK_, DLO_), jnp.float32),
            pltpu.VMEM((ZROWS_, DLO_), jnp.float32),
            pltpu.VMEM_SHARED((NPAD_, DLO_), jnp.float32),
            pltpu.SemaphoreType.DMA,
        ],
    )
    def k(sup_lo_hbm, sup_hi_hbm, row_hbm, col_hbm, typ_hbm, arows_hbm,
          out_lo_hbm, out_hi_hbm,
          rows_v, src_v, dst_v, typ_v, arow_v, zero_v, acc, sem):
        c = lax.axis_index("c")
        s = lax.axis_index("s")
        zvec = jnp.zeros((16,), jnp.float32)

        def zbody(i, _):
            def zcol(kk, __):
                zero_v[i, pl.ds(kk * 16, 16)] = zvec
                return __
            return lax.fori_loop(0, DLO_ // 16, zcol, _)
        lax.fori_loop(0, ZROWS_, zbody, 0)

        for piece in range(ROWS_PER_TILE_ // ZROWS_):
            pltpu.sync_copy(
                zero_v,
                acc.at[pl.ds(s * ROWS_PER_TILE_ + piece * ZROWS_, ZROWS_)])
        plsc.subcore_barrier()

        def do_chunk(base, src_hbm, dst_hbm):
            pltpu.sync_copy(src_hbm.at[pl.ds(base, CHUNK_)], src_v)
            pltpu.sync_copy(dst_hbm.at[pl.ds(base, CHUNK_)], dst_v)
            pltpu.sync_copy(typ_hbm.at[pl.ds(base, CHUNK_)], typ_v)

            @pl.when(c == 0)
            def _():
                pltpu.make_async_copy(sup_lo_hbm.at[src_v, :], rows_v, sem).start()

            @pl.when(c == 1)
            def _():
                pltpu.make_async_copy(sup_hi_hbm.at[src_v, :], rows_v, sem).start()

            acp = pltpu.make_async_copy(arows_hbm.at[typ_v, :], arow_v, sem)
            acp.start()
            acp.wait()
            acp.wait()

            def scale(e, _):
                for kk in range(DLO_ // 16):
                    sl = pl.ds(kk * 16, 16)
                    rows_v[e, sl] = rows_v[e, sl] * arow_v[e, sl]
                return _
            lax.fori_loop(0, CHUNK_, scale, 0)

            pltpu.sync_copy(rows_v, acc.at[dst_v, :], add=True)

        stripe0 = s * STRIPE_

        def chunk_loop(i, _):
            do_chunk(stripe0 + i * CHUNK_, col_hbm, row_hbm)
            return _
        lax.fori_loop(0, NCHUNK_, chunk_loop, 0)

        def chunk_loop_t(i, _):
            do_chunk(stripe0 + i * CHUNK_, row_hbm, col_hbm)
            return _
        lax.fori_loop(0, NCHUNK_, chunk_loop_t, 0)

        plsc.subcore_barrier()

        @pl.when(c == 0)
        def _():
            pltpu.sync_copy(acc.at[pl.ds(s * ROWS_PER_TILE_, ROWS_PER_TILE_)],
                            out_lo_hbm.at[pl.ds(s * ROWS_PER_TILE_, ROWS_PER_TILE_)])

        @pl.when(c == 1)
        def _():
            pltpu.sync_copy(acc.at[pl.ds(s * ROWS_PER_TILE_, ROWS_PER_TILE_)],
                            out_hi_hbm.at[pl.ds(s * ROWS_PER_TILE_, ROWS_PER_TILE_)])

    return k(sup_lo, sup_hi, row, col, etype, arows)


_HI = lax.Precision.HIGHEST


def _tc_pre(emb_pad, gc1_w):
    def body(x_ref, w_ref, o_ref):
        o_ref[...] = jnp.dot(x_ref[...], w_ref[...],
                             preferred_element_type=jnp.float32, precision=_HI)
    kdim = emb_pad.shape[1]
    return pl.pallas_call(
        body,
        grid=(NRB_,),
        in_specs=[pl.BlockSpec((RB_, kdim), lambda i: (i, 0)),
                  pl.BlockSpec((kdim, D_), lambda i: (0, 0))],
        out_specs=pl.BlockSpec((RB_, D_), lambda i: (i, 0)),
        out_shape=jax.ShapeDtypeStruct((NPAD_, D_), jnp.float32),
    )(emb_pad, gc1_w)


def _tc_colstats(x):
    def body(x_ref, s1_ref, s2_ref):
        i = pl.program_id(0)
        rowid = i * RB_ + lax.broadcasted_iota(jnp.int32, (RB_, 1), 0)
        valid = rowid < N_ENT_
        xb = jnp.where(valid, x_ref[...], 0.0)
        s1_ref[...] = xb.sum(axis=0)[None, None, :]
        s2_ref[...] = (xb * xb).sum(axis=0)[None, None, :]
    return pl.pallas_call(
        body,
        grid=(NRB_,),
        in_specs=[pl.BlockSpec((RB_, D_), lambda i: (i, 0))],
        out_specs=[pl.BlockSpec((1, 1, D_), lambda i: (i, 0, 0)),
                   pl.BlockSpec((1, 1, D_), lambda i: (i, 0, 0))],
        out_shape=[jax.ShapeDtypeStruct((NRB_, 1, D_), jnp.float32),
                   jax.ShapeDtypeStruct((NRB_, 1, D_), jnp.float32)],
    )(x)


def _bn_affine(s1, s2, g, b):
    m = s1.sum(axis=(0, 1)) / N_ENT_
    msq = s2.sum(axis=(0, 1)) / N_ENT_
    var = msq - m * m
    scale = g / jnp.sqrt(var + EPS_)
    return scale, b - m * scale


def _tc_mid(agg1, s1, s2, bn3_g, bn3_b, gc2_w):
    def body(a_ref, s1_ref, s2_ref, g_ref, b_ref, w_ref, o_ref):
        scale, shift = _bn_affine(s1_ref[...], s2_ref[...], g_ref[...], b_ref[...])
        x = jnp.tanh(a_ref[...] * scale + shift)
        o_ref[...] = jnp.dot(x, w_ref[...],
                             preferred_element_type=jnp.float32, precision=_HI)
    return pl.pallas_call(
        body,
        grid=(NRB_,),
        in_specs=[pl.BlockSpec((RB_, D_), lambda i: (i, 0)),
                  pl.BlockSpec((NRB_, 1, D_), lambda i: (0, 0, 0)),
                  pl.BlockSpec((NRB_, 1, D_), lambda i: (0, 0, 0)),
                  pl.BlockSpec((D_,), lambda i: (0,)),
                  pl.BlockSpec((D_,), lambda i: (0,)),
                  pl.BlockSpec((D_, D_), lambda i: (0, 0))],
        out_specs=pl.BlockSpec((RB_, D_), lambda i: (i, 0)),
        out_shape=jax.ShapeDtypeStruct((NPAD_, D_), jnp.float32),
    )(agg1, s1, s2, bn3_g, bn3_b, gc2_w)


def _tc_eall(agg2, s1, s2, bn4_g, bn4_b):
    def body(a_ref, s1_ref, s2_ref, g_ref, b_ref, o_ref):
        scale, shift = _bn_affine(s1_ref[...], s2_ref[...], g_ref[...], b_ref[...])
        o_ref[...] = jnp.tanh(a_ref[...] * scale + shift)
    return pl.pallas_call(
        body,
        grid=(NRB_,),
        in_specs=[pl.BlockSpec((RB_, D_), lambda i: (i, 0)),
                  pl.BlockSpec((NRB_, 1, D_), lambda i: (0, 0, 0)),
                  pl.BlockSpec((NRB_, 1, D_), lambda i: (0, 0, 0)),
                  pl.BlockSpec((D_,), lambda i: (0,)),
                  pl.BlockSpec((D_,), lambda i: (0,))],
        out_specs=pl.BlockSpec((RB_, D_), lambda i: (i, 0)),
        out_shape=jax.ShapeDtypeStruct((NPAD_, D_), jnp.float32),
    )(agg2, s1, s2, bn4_g, bn4_b)


def _tc_gather_e1(e_all, e1):
    B = e1.shape[0]

    def body(ea_ref, e1_ref, o_ref):
        i = pl.program_id(0)

        @pl.when(i == 0)
        def _():
            o_ref[...] = jnp.zeros_like(o_ref)

        ids = i * RB_ + lax.broadcasted_iota(jnp.int32, (1, RB_), 1)
        oh = (e1_ref[...] == ids).astype(jnp.float32)
        o_ref[...] += jnp.dot(oh, ea_ref[...],
                              preferred_element_type=jnp.float32, precision=_HI)

    return pl.pallas_call(
        body,
        grid=(NRB_,),
        in_specs=[pl.BlockSpec((RB_, D_), lambda i: (i, 0)),
                  pl.BlockSpec((B, 1), lambda i: (0, 0))],
        out_specs=pl.BlockSpec((B, D_), lambda i: (0, 0)),
        out_shape=jax.ShapeDtypeStruct((B, D_), jnp.float32),
    )(e_all, e1)


def _tc_conv(e1_emb, rel, emb_rel, conv_w, cb2, g1r, b1r, bn0_g, bn0_b):
    B = e1_emb.shape[0]
    CH = conv_w.shape[0]
    CB = 40
    NR = emb_rel.shape[0]

    def body(e1e_ref, rel_ref, er_ref, cw_ref, cb_ref, g1_ref, b1_ref,
             g0_ref, b0_ref, o_ref):
        e1_emb_v = e1e_ref[...]
        rids = lax.broadcasted_iota(jnp.int32, (1, NR), 1)
        oh_rel = (rel_ref[...] == rids).astype(jnp.float32)
        rel_emb = jnp.dot(oh_rel, er_ref[...],
                          preferred_element_type=jnp.float32, precision=_HI)

        m0 = e1_emb_v.mean(); v0 = ((e1_emb_v - m0) ** 2).mean()
        z0 = (e1_emb_v - m0) / jnp.sqrt(v0 + EPS_) * g0_ref[0] + b0_ref[0]
        m1 = rel_emb.mean(); v1 = ((rel_emb - m1) ** 2).mean()
        z1 = (rel_emb - m1) / jnp.sqrt(v1 + EPS_) * g0_ref[1] + b0_ref[1]

        zp = jnp.zeros((B, 2), jnp.float32)
        xp0 = jnp.concatenate([zp, z0, zp], axis=1)
        xp1 = jnp.concatenate([zp, z1, zp], axis=1)
        cw = cw_ref[...]
        acc = jnp.zeros((B, CB, D_), jnp.float32)
        for kk in range(5):
            acc = acc + xp0[:, kk:kk + D_][:, None, :] * cw[None, :, 0, kk][:, :, None]
            acc = acc + xp1[:, kk:kk + D_][:, None, :] * cw[None, :, 1, kk][:, :, None]
        acc = acc + cb_ref[...][None, :, 0][:, :, None]

        mc = acc.mean(axis=(0, 2), keepdims=True)
        vc = ((acc - mc) ** 2).mean(axis=(0, 2), keepdims=True)
        acc = (acc - mc) / jnp.sqrt(vc + EPS_) * g1_ref[...][None, :, 0][:, :, None] \
            + b1_ref[...][None, :, 0][:, :, None]
        o_ref[...] = jnp.maximum(acc, 0.0)

    return pl.pallas_call(
        body,
        grid=(CH // CB,),
        in_specs=[pl.BlockSpec((B, D_), lambda i: (0, 0)),
                  pl.BlockSpec((B, 1), lambda i: (0, 0)),
                  pl.BlockSpec((NR, D_), lambda i: (0, 0)),
                  pl.BlockSpec((CB, 2, 5), lambda i: (i, 0, 0)),
                  pl.BlockSpec((CB, 1), lambda i: (i, 0)),
                  pl.BlockSpec((CB, 1), lambda i: (i, 0)),
                  pl.BlockSpec((CB, 1), lambda i: (i, 0)),
                  pl.BlockSpec((2,), lambda i: (0,)),
                  pl.BlockSpec((2,), lambda i: (0,))],
        out_specs=pl.BlockSpec((B, CB, D_), lambda i: (0, i, 0)),
        out_shape=jax.ShapeDtypeStruct((B, CH, D_), jnp.float32),
    )(e1_emb, rel, emb_rel, conv_w, cb2, g1r, b1r, bn0_g, bn0_b)


def _tc_fc(x2, fw):
    B = x2.shape[0]
    K = x2.shape[1]
    NB = 8
    KB = K // NB

    def body(x_ref, w_ref, o_ref):
        i = pl.program_id(0)

        @pl.when(i == 0)
        def _():
            o_ref[...] = jnp.zeros_like(o_ref)

        o_ref[...] += lax.dot_general(
            x_ref[...], w_ref[...], (((1,), (1,)), ((), ())),
            preferred_element_type=jnp.float32, precision=_HI)

    return pl.pallas_call(
        body,
        grid=(NB,),
        in_specs=[pl.BlockSpec((B, KB), lambda i: (0, i)),
                  pl.BlockSpec((D_, KB), lambda i: (0, i))],
        out_specs=pl.BlockSpec((B, D_), lambda i: (0, 0)),
        out_shape=jax.ShapeDtypeStruct((B, D_), jnp.float32),
    )(x2, fw)


def _tc_logits(y, fc_b, bn2_g, bn2_b, e_all):
    B = y.shape[0]

    def body(y_ref, fb_ref, g2_ref, b2_ref, ea_ref, o_ref):
        y2 = y_ref[...] + fb_ref[...]
        m = y2.mean(axis=0)
        v = ((y2 - m) ** 2).mean(axis=0)
        y2 = (y2 - m) / jnp.sqrt(v + EPS_) * g2_ref[...] + b2_ref[...]
        y2 = jnp.maximum(y2, 0.0)
        logits = lax.dot_general(y2, ea_ref[...], (((1,), (1,)), ((), ())),
                                 preferred_element_type=jnp.float32, precision=_HI)
        o_ref[...] = jax.nn.sigmoid(logits)

    return pl.pallas_call(
        body,
        grid=(NRB_,),
        in_specs=[pl.BlockSpec((B, D_), lambda i: (0, 0)),
                  pl.BlockSpec((D_,), lambda i: (0,)),
                  pl.BlockSpec((D_,), lambda i: (0,)),
                  pl.BlockSpec((D_,), lambda i: (0,)),
                  pl.BlockSpec((RB_, D_), lambda i: (i, 0))],
        out_specs=pl.BlockSpec((B, RB_), lambda i: (0, i)),
        out_shape=jax.ShapeDtypeStruct((B, NPAD_), jnp.float32),
    )(y, fc_b, bn2_g, bn2_b, e_all)


def _split_pad(sup):
    z = jnp.zeros((sup.shape[0], DLO_ - DHALF_), sup.dtype)
    lo = jnp.concatenate([sup[:, :DHALF_], z], axis=1)
    hi = jnp.concatenate([sup[:, DHALF_:], z], axis=1)
    return lo, hi


def _assemble(lo, hi):
    return jnp.concatenate([lo[:, :DHALF_], hi[:, :DHALF_]], axis=1)


def kernel(e1, rel, X, edge_index, edge_type, num_nodes, emb_e, gc1_w, gc1_b, gc1_alpha, gc2_w, gc2_b, gc2_alpha, emb_rel, conv_w, conv_b, fc_w, fc_b, bn0_g, bn0_b, bn1_g, bn1_b, bn2_g, bn2_b, bn3_g, bn3_b, bn4_g, bn4_b):
    B = e1.shape[0]
    row = edge_index[0].astype(jnp.int32)
    col = edge_index[1].astype(jnp.int32)
    etype = edge_type.astype(jnp.int32)
    a1 = jnp.concatenate([gc1_alpha[:, 0], jnp.zeros((512 - gc1_alpha.shape[0],), jnp.float32)])
    a2 = jnp.concatenate([gc2_alpha[:, 0], jnp.zeros((512 - gc2_alpha.shape[0],), jnp.float32)])
    a1_rows = jnp.broadcast_to(a1[:, None], (512, DLO_))
    a2_rows = jnp.broadcast_to(a2[:, None], (512, DLO_))
    emb_pad = jnp.concatenate(
        [emb_e, jnp.zeros((NPAD_ - N_ENT_, emb_e.shape[1]), jnp.float32)], axis=0)

    sup1 = _tc_pre(emb_pad, gc1_w)
    s1_lo, s1_hi = _split_pad(sup1)
    g1_lo, g1_hi = _sc_agg(s1_lo, s1_hi, row, col, etype, a1_rows)
    agg1 = _assemble(g1_lo, g1_hi)

    st1a, st1b = _tc_colstats(agg1)
    sup2 = _tc_mid(agg1, st1a, st1b, bn3_g, bn3_b, gc2_w)
    s2_lo, s2_hi = _split_pad(sup2)
    g2_lo, g2_hi = _sc_agg(s2_lo, s2_hi, row, col, etype, a2_rows)
    agg2 = _assemble(g2_lo, g2_hi)

    st2a, st2b = _tc_colstats(agg2)
    e_all = _tc_eall(agg2, st2a, st2b, bn4_g, bn4_b)
    e1_emb = _tc_gather_e1(e_all, e1.astype(jnp.int32))

    cb2 = conv_b.reshape(-1, 1)
    g1r = bn1_g.reshape(-1, 1)
    b1r = bn1_b.reshape(-1, 1)
    h = _tc_conv(e1_emb, rel.astype(jnp.int32), emb_rel, conv_w, cb2, g1r, b1r,
                 bn0_g, bn0_b)

    x2 = h.reshape(B, -1)
    kpad = 40960 - x2.shape[1]
    x2p = jnp.concatenate([x2, jnp.zeros((B, kpad), jnp.float32)], axis=1)
    fwp = jnp.concatenate([fc_w, jnp.zeros((fc_w.shape[0], kpad), jnp.float32)], axis=1)
    y = _tc_fc(x2p, fwp)

    out_pad = _tc_logits(y, fc_b, bn2_g, bn2_b, e_all)
    return out_pad[:, :N_ENT_]


# speedup vs baseline: 3.7994x; 1.4834x over previous
import functools
import jax
import jax.numpy as jnp
from jax import lax
from jax.experimental import pallas as pl
from jax.experimental.pallas import tpu as pltpu
from jax.experimental.pallas import tpu_sc as plsc

N_ENT_ = 10000
NPAD_ = 10240
D_ = 200
DLO_ = 128
DHALF_ = 100
E_ = 320000
CHUNK_ = 80
STRIPE_ = E_ // 16
NCHUNK_ = STRIPE_ // CHUNK_
ZROWS_ = 32
ROWS_PER_TILE_ = NPAD_ // 16
RB_ = 1280
NRB_ = NPAD_ // RB_
EPS_ = 1e-5


def _sc_agg(sup_lo, sup_hi, row, col, etype, arows):
    mesh = plsc.VectorSubcoreMesh(core_axis_name="c", subcore_axis_name="s")

    @functools.partial(
        pl.kernel,
        mesh=mesh,
        out_type=[
            jax.ShapeDtypeStruct((NPAD_, DLO_), jnp.float32),
            jax.ShapeDtypeStruct((NPAD_, DLO_), jnp.float32),
        ],
        scratch_types=(
            [pltpu.VMEM((CHUNK_, DLO_), jnp.float32) for _ in range(2)] +
            [pltpu.VMEM((CHUNK_, DLO_), jnp.float32) for _ in range(2)] +
            [pltpu.VMEM((CHUNK_,), jnp.int32) for _ in range(2)] +
            [pltpu.VMEM((CHUNK_,), jnp.int32) for _ in range(2)] +
            [pltpu.VMEM((CHUNK_,), jnp.int32) for _ in range(2)] +
            [pltpu.VMEM((ZROWS_, DLO_), jnp.float32),
             pltpu.VMEM_SHARED((NPAD_, DLO_), jnp.float32)] +
            [pltpu.SemaphoreType.DMA for _ in range(2)]
        ),
    )
    def k(sup_lo_hbm, sup_hi_hbm, row_hbm, col_hbm, typ_hbm, arows_hbm,
          out_lo_hbm, out_hi_hbm,
          r0, r1, a0, a1, s0, s1, d0, d1, t0, t1, zero_v, acc, g0, g1):
        rows_b = [r0, r1]
        arow_b = [a0, a1]
        src_b = [s0, s1]
        dst_b = [d0, d1]
        typ_b = [t0, t1]
        semg_b = [g0, g1]
        c = lax.axis_index("c")
        s = lax.axis_index("s")
        zvec = jnp.zeros((16,), jnp.float32)

        def zbody(i, _):
            def zcol(kk, __):
                zero_v[i, pl.ds(kk * 16, 16)] = zvec
                return __
            return lax.fori_loop(0, DLO_ // 16, zcol, _)
        lax.fori_loop(0, ZROWS_, zbody, 0)

        for piece in range(ROWS_PER_TILE_ // ZROWS_):
            pltpu.sync_copy(
                zero_v,
                acc.at[pl.ds(s * ROWS_PER_TILE_ + piece * ZROWS_, ZROWS_)])
        plsc.subcore_barrier()

        stripe0 = s * STRIPE_
        J = 2 * NCHUNK_

        def issue(j, b):
            @pl.when(j < NCHUNK_)
            def _():
                base = stripe0 + j * CHUNK_
                pltpu.sync_copy(col_hbm.at[pl.ds(base, CHUNK_)], src_b[b])
                pltpu.sync_copy(row_hbm.at[pl.ds(base, CHUNK_)], dst_b[b])
                pltpu.sync_copy(typ_hbm.at[pl.ds(base, CHUNK_)], typ_b[b])

            @pl.when(j >= NCHUNK_)
            def _():
                base = stripe0 + (j - NCHUNK_) * CHUNK_
                pltpu.sync_copy(row_hbm.at[pl.ds(base, CHUNK_)], src_b[b])
                pltpu.sync_copy(col_hbm.at[pl.ds(base, CHUNK_)], dst_b[b])
                pltpu.sync_copy(typ_hbm.at[pl.ds(base, CHUNK_)], typ_b[b])

            @pl.when(c == 0)
            def _():
                pltpu.make_async_copy(
                    sup_lo_hbm.at[src_b[b], :], rows_b[b], semg_b[b]).start()

            @pl.when(c == 1)
            def _():
                pltpu.make_async_copy(
                    sup_hi_hbm.at[src_b[b], :], rows_b[b], semg_b[b]).start()

            pltpu.make_async_copy(
                arows_hbm.at[typ_b[b], :], arow_b[b], semg_b[b]).start()

        for b in range(2):
            issue(b, b)

        def iter_body(i, car):
            for b in range(2):
                j = i * 2 + b
                gw = pltpu.make_async_copy(
                    arows_hbm.at[typ_b[b], :], arow_b[b], semg_b[b])
                gw.wait()
                gw.wait()

                def scale(e, __):
                    for kk in range(DLO_ // 16):
                        sl = pl.ds(kk * 16, 16)
                        rows_b[b][e, sl] = rows_b[b][e, sl] * arow_b[b][e, sl]
                    return __
                lax.fori_loop(0, CHUNK_, scale, 0)

                pltpu.sync_copy(rows_b[b], acc.at[dst_b[b], :], add=True)

                jn = j + 2

                @pl.when(jn < J)
                def _():
                    issue(jn, b)
            return car
        lax.fori_loop(0, J // 2, iter_body, 0)

        plsc.subcore_barrier()

        @pl.when(c == 0)
        def _():
            pltpu.sync_copy(acc.at[pl.ds(s * ROWS_PER_TILE_, ROWS_PER_TILE_)],
                            out_lo_hbm.at[pl.ds(s * ROWS_PER_TILE_, ROWS_PER_TILE_)])

        @pl.when(c == 1)
        def _():
            pltpu.sync_copy(acc.at[pl.ds(s * ROWS_PER_TILE_, ROWS_PER_TILE_)],
                            out_hi_hbm.at[pl.ds(s * ROWS_PER_TILE_, ROWS_PER_TILE_)])

    return k(sup_lo, sup_hi, row, col, etype, arows)


_HI = lax.Precision.HIGHEST


def _tc_pre(emb_pad, gc1_w):
    def body(x_ref, w_ref, o_ref):
        o_ref[...] = jnp.dot(x_ref[...], w_ref[...],
                             preferred_element_type=jnp.float32, precision=_HI)
    kdim = emb_pad.shape[1]
    return pl.pallas_call(
        body,
        grid=(NRB_,),
        in_specs=[pl.BlockSpec((RB_, kdim), lambda i: (i, 0)),
                  pl.BlockSpec((kdim, D_), lambda i: (0, 0))],
        out_specs=pl.BlockSpec((RB_, D_), lambda i: (i, 0)),
        out_shape=jax.ShapeDtypeStruct((NPAD_, D_), jnp.float32),
    )(emb_pad, gc1_w)


def _tc_colstats(x):
    def body(x_ref, s1_ref, s2_ref):
        i = pl.program_id(0)
        rowid = i * RB_ + lax.broadcasted_iota(jnp.int32, (RB_, 1), 0)
        valid = rowid < N_ENT_
        xb = jnp.where(valid, x_ref[...], 0.0)
        s1_ref[...] = xb.sum(axis=0)[None, None, :]
        s2_ref[...] = (xb * xb).sum(axis=0)[None, None, :]
    return pl.pallas_call(
        body,
        grid=(NRB_,),
        in_specs=[pl.BlockSpec((RB_, D_), lambda i: (i, 0))],
        out_specs=[pl.BlockSpec((1, 1, D_), lambda i: (i, 0, 0)),
                   pl.BlockSpec((1, 1, D_), lambda i: (i, 0, 0))],
        out_shape=[jax.ShapeDtypeStruct((NRB_, 1, D_), jnp.float32),
                   jax.ShapeDtypeStruct((NRB_, 1, D_), jnp.float32)],
    )(x)


def _bn_affine(s1, s2, g, b):
    m = s1.sum(axis=(0, 1)) / N_ENT_
    msq = s2.sum(axis=(0, 1)) / N_ENT_
    var = msq - m * m
    scale = g / jnp.sqrt(var + EPS_)
    return scale, b - m * scale


def _tc_mid(agg1, s1, s2, bn3_g, bn3_b, gc2_w):
    def body(a_ref, s1_ref, s2_ref, g_ref, b_ref, w_ref, o_ref):
        scale, shift = _bn_affine(s1_ref[...], s2_ref[...], g_ref[...], b_ref[...])
        x = jnp.tanh(a_ref[...] * scale + shift)
        o_ref[...] = jnp.dot(x, w_ref[...],
                             preferred_element_type=jnp.float32, precision=_HI)
    return pl.pallas_call(
        body,
        grid=(NRB_,),
        in_specs=[pl.BlockSpec((RB_, D_), lambda i: (i, 0)),
                  pl.BlockSpec((NRB_, 1, D_), lambda i: (0, 0, 0)),
                  pl.BlockSpec((NRB_, 1, D_), lambda i: (0, 0, 0)),
                  pl.BlockSpec((D_,), lambda i: (0,)),
                  pl.BlockSpec((D_,), lambda i: (0,)),
                  pl.BlockSpec((D_, D_), lambda i: (0, 0))],
        out_specs=pl.BlockSpec((RB_, D_), lambda i: (i, 0)),
        out_shape=jax.ShapeDtypeStruct((NPAD_, D_), jnp.float32),
    )(agg1, s1, s2, bn3_g, bn3_b, gc2_w)


def _tc_eall(agg2, s1, s2, bn4_g, bn4_b):
    def body(a_ref, s1_ref, s2_ref, g_ref, b_ref, o_ref):
        scale, shift = _bn_affine(s1_ref[...], s2_ref[...], g_ref[...], b_ref[...])
        o_ref[...] = jnp.tanh(a_ref[...] * scale + shift)
    return pl.pallas_call(
        body,
        grid=(NRB_,),
        in_specs=[pl.BlockSpec((RB_, D_), lambda i: (i, 0)),
                  pl.BlockSpec((NRB_, 1, D_), lambda i: (0, 0, 0)),
                  pl.BlockSpec((NRB_, 1, D_), lambda i: (0, 0, 0)),
                  pl.BlockSpec((D_,), lambda i: (0,)),
                  pl.BlockSpec((D_,), lambda i: (0,))],
        out_specs=pl.BlockSpec((RB_, D_), lambda i: (i, 0)),
        out_shape=jax.ShapeDtypeStruct((NPAD_, D_), jnp.float32),
    )(agg2, s1, s2, bn4_g, bn4_b)


def _tc_gather_e1(e_all, e1):
    B = e1.shape[0]

    def body(ea_ref, e1_ref, o_ref):
        i = pl.program_id(0)

        @pl.when(i == 0)
        def _():
            o_ref[...] = jnp.zeros_like(o_ref)

        ids = i * RB_ + lax.broadcasted_iota(jnp.int32, (1, RB_), 1)
        oh = (e1_ref[...] == ids).astype(jnp.float32)
        o_ref[...] += jnp.dot(oh, ea_ref[...],
                              preferred_element_type=jnp.float32, precision=_HI)

    return pl.pallas_call(
        body,
        grid=(NRB_,),
        in_specs=[pl.BlockSpec((RB_, D_), lambda i: (i, 0)),
                  pl.BlockSpec((B, 1), lambda i: (0, 0))],
        out_specs=pl.BlockSpec((B, D_), lambda i: (0, 0)),
        out_shape=jax.ShapeDtypeStruct((B, D_), jnp.float32),
    )(e_all, e1)


def _tc_conv(e1_emb, rel, emb_rel, conv_w, cb2, g1r, b1r, bn0_g, bn0_b):
    B = e1_emb.shape[0]
    CH = conv_w.shape[0]
    CB = 40
    NR = emb_rel.shape[0]

    def body(e1e_ref, rel_ref, er_ref, cw_ref, cb_ref, g1_ref, b1_ref,
             g0_ref, b0_ref, o_ref):
        e1_emb_v = e1e_ref[...]
        rids = lax.broadcasted_iota(jnp.int32, (1, NR), 1)
        oh_rel = (rel_ref[...] == rids).astype(jnp.float32)
        rel_emb = jnp.dot(oh_rel, er_ref[...],
                          preferred_element_type=jnp.float32, precision=_HI)

        m0 = e1_emb_v.mean(); v0 = ((e1_emb_v - m0) ** 2).mean()
        z0 = (e1_emb_v - m0) / jnp.sqrt(v0 + EPS_) * g0_ref[0] + b0_ref[0]
        m1 = rel_emb.mean(); v1 = ((rel_emb - m1) ** 2).mean()
        z1 = (rel_emb - m1) / jnp.sqrt(v1 + EPS_) * g0_ref[1] + b0_ref[1]

        zp = jnp.zeros((B, 2), jnp.float32)
        xp0 = jnp.concatenate([zp, z0, zp], axis=1)
        xp1 = jnp.concatenate([zp, z1, zp], axis=1)
        cw = cw_ref[...]
        acc = jnp.zeros((B, CB, D_), jnp.float32)
        for kk in range(5):
            acc = acc + xp0[:, kk:kk + D_][:, None, :] * cw[None, :, 0, kk][:, :, None]
            acc = acc + xp1[:, kk:kk + D_][:, None, :] * cw[None, :, 1, kk][:, :, None]
        acc = acc + cb_ref[...][None, :, 0][:, :, None]

        mc = acc.mean(axis=(0, 2), keepdims=True)
        vc = ((acc - mc) ** 2).mean(axis=(0, 2), keepdims=True)
        acc = (acc - mc) / jnp.sqrt(vc + EPS_) * g1_ref[...][None, :, 0][:, :, None] \
            + b1_ref[...][None, :, 0][:, :, None]
        o_ref[...] = jnp.maximum(acc, 0.0)

    return pl.pallas_call(
        body,
        grid=(CH // CB,),
        in_specs=[pl.BlockSpec((B, D_), lambda i: (0, 0)),
                  pl.BlockSpec((B, 1), lambda i: (0, 0)),
                  pl.BlockSpec((NR, D_), lambda i: (0, 0)),
                  pl.BlockSpec((CB, 2, 5), lambda i: (i, 0, 0)),
                  pl.BlockSpec((CB, 1), lambda i: (i, 0)),
                  pl.BlockSpec((CB, 1), lambda i: (i, 0)),
                  pl.BlockSpec((CB, 1), lambda i: (i, 0)),
                  pl.BlockSpec((2,), lambda i: (0,)),
                  pl.BlockSpec((2,), lambda i: (0,))],
        out_specs=pl.BlockSpec((B, CB, D_), lambda i: (0, i, 0)),
        out_shape=jax.ShapeDtypeStruct((B, CH, D_), jnp.float32),
    )(e1_emb, rel, emb_rel, conv_w, cb2, g1r, b1r, bn0_g, bn0_b)


def _tc_fc(x2, fw):
    B = x2.shape[0]
    K = x2.shape[1]
    NB = 8
    KB = K // NB

    def body(x_ref, w_ref, o_ref):
        i = pl.program_id(0)

        @pl.when(i == 0)
        def _():
            o_ref[...] = jnp.zeros_like(o_ref)

        o_ref[...] += lax.dot_general(
            x_ref[...], w_ref[...], (((1,), (1,)), ((), ())),
            preferred_element_type=jnp.float32, precision=_HI)

    return pl.pallas_call(
        body,
        grid=(NB,),
        in_specs=[pl.BlockSpec((B, KB), lambda i: (0, i)),
                  pl.BlockSpec((D_, KB), lambda i: (0, i))],
        out_specs=pl.BlockSpec((B, D_), lambda i: (0, 0)),
        out_shape=jax.ShapeDtypeStruct((B, D_), jnp.float32),
    )(x2, fw)


def _tc_logits(y, fc_b, bn2_g, bn2_b, e_all):
    B = y.shape[0]

    def body(y_ref, fb_ref, g2_ref, b2_ref, ea_ref, o_ref):
        y2 = y_ref[...] + fb_ref[...]
        m = y2.mean(axis=0)
        v = ((y2 - m) ** 2).mean(axis=0)
        y2 = (y2 - m) / jnp.sqrt(v + EPS_) * g2_ref[...] + b2_ref[...]
        y2 = jnp.maximum(y2, 0.0)
        logits = lax.dot_general(y2, ea_ref[...], (((1,), (1,)), ((), ())),
                                 preferred_element_type=jnp.float32, precision=_HI)
        o_ref[...] = jax.nn.sigmoid(logits)

    return pl.pallas_call(
        body,
        grid=(NRB_,),
        in_specs=[pl.BlockSpec((B, D_), lambda i: (0, 0)),
                  pl.BlockSpec((D_,), lambda i: (0,)),
                  pl.BlockSpec((D_,), lambda i: (0,)),
                  pl.BlockSpec((D_,), lambda i: (0,)),
                  pl.BlockSpec((RB_, D_), lambda i: (i, 0))],
        out_specs=pl.BlockSpec((B, RB_), lambda i: (0, i)),
        out_shape=jax.ShapeDtypeStruct((B, NPAD_), jnp.float32),
    )(y, fc_b, bn2_g, bn2_b, e_all)


def _split_pad(sup):
    z = jnp.zeros((sup.shape[0], DLO_ - DHALF_), sup.dtype)
    lo = jnp.concatenate([sup[:, :DHALF_], z], axis=1)
    hi = jnp.concatenate([sup[:, DHALF_:], z], axis=1)
    return lo, hi


def _assemble(lo, hi):
    return jnp.concatenate([lo[:, :DHALF_], hi[:, :DHALF_]], axis=1)


def kernel(e1, rel, X, edge_index, edge_type, num_nodes, emb_e, gc1_w, gc1_b, gc1_alpha, gc2_w, gc2_b, gc2_alpha, emb_rel, conv_w, conv_b, fc_w, fc_b, bn0_g, bn0_b, bn1_g, bn1_b, bn2_g, bn2_b, bn3_g, bn3_b, bn4_g, bn4_b):
    B = e1.shape[0]
    row = edge_index[0].astype(jnp.int32)
    col = edge_index[1].astype(jnp.int32)
    etype = edge_type.astype(jnp.int32)
    a1 = jnp.concatenate([gc1_alpha[:, 0], jnp.zeros((512 - gc1_alpha.shape[0],), jnp.float32)])
    a2 = jnp.concatenate([gc2_alpha[:, 0], jnp.zeros((512 - gc2_alpha.shape[0],), jnp.float32)])
    a1_rows = jnp.broadcast_to(a1[:, None], (512, DLO_))
    a2_rows = jnp.broadcast_to(a2[:, None], (512, DLO_))
    emb_pad = jnp.concatenate(
        [emb_e, jnp.zeros((NPAD_ - N_ENT_, emb_e.shape[1]), jnp.float32)], axis=0)

    sup1 = _tc_pre(emb_pad, gc1_w)
    s1_lo, s1_hi = _split_pad(sup1)
    g1_lo, g1_hi = _sc_agg(s1_lo, s1_hi, row, col, etype, a1_rows)
    agg1 = _assemble(g1_lo, g1_hi)

    st1a, st1b = _tc_colstats(agg1)
    sup2 = _tc_mid(agg1, st1a, st1b, bn3_g, bn3_b, gc2_w)
    s2_lo, s2_hi = _split_pad(sup2)
    g2_lo, g2_hi = _sc_agg(s2_lo, s2_hi, row, col, etype, a2_rows)
    agg2 = _assemble(g2_lo, g2_hi)

    st2a, st2b = _tc_colstats(agg2)
    e_all = _tc_eall(agg2, st2a, st2b, bn4_g, bn4_b)
    e1_emb = _tc_gather_e1(e_all, e1.astype(jnp.int32))

    cb2 = conv_b.reshape(-1, 1)
    g1r = bn1_g.reshape(-1, 1)
    b1r = bn1_b.reshape(-1, 1)
    h = _tc_conv(e1_emb, rel.astype(jnp.int32), emb_rel, conv_w, cb2, g1r, b1r,
                 bn0_g, bn0_b)

    x2 = h.reshape(B, -1)
    kpad = 40960 - x2.shape[1]
    x2p = jnp.concatenate([x2, jnp.zeros((B, kpad), jnp.float32)], axis=1)
    fwp = jnp.concatenate([fc_w, jnp.zeros((fc_w.shape[0], kpad), jnp.float32)], axis=1)
    y = _tc_fc(x2p, fwp)

    out_pad = _tc_logits(y, fc_b, bn2_g, bn2_b, e_all)
    return out_pad[:, :N_ENT_]
